# baseline (speedup 1.0000x reference)
import dataclasses
import functools

import jax
import jax.numpy as jnp
from jax import lax
from jax.experimental import pallas as pl
from jax.experimental.pallas import tpu as pltpu
from jax.experimental.pallas import tpu_sc as plsc

_NW = 32
_CHR = 8


def kernel(x, pos_table):
    B, L, D = x.shape
    LD = L * D
    x2 = jnp.transpose(x, (1, 2, 0)).reshape(LD, B)
    pos_flat = jax.lax.slice(pos_table, (0, 0), (L, D)).reshape(LD)
    rows_per_w = LD // _NW
    n_chunks = rows_per_w // _CHR
    mesh = plsc.VectorSubcoreMesh(core_axis_name="c", subcore_axis_name="s")

    cp = pltpu.CompilerParams()
    if "needs_layout_passes" in pltpu.CompilerParams.__dataclass_fields__:
        cp = dataclasses.replace(cp, needs_layout_passes=False)

    @functools.partial(
        pl.kernel,
        mesh=mesh,
        compiler_params=cp,
        out_type=jax.ShapeDtypeStruct((LD, B), jnp.float32),
        scratch_types=[
            pltpu.VMEM((_CHR, B), jnp.float32),
            pltpu.VMEM((rows_per_w,), jnp.float32),
        ],
    )
    def sc_add(x_hbm, p_hbm, o_hbm, xbuf, pbuf):
        wid = lax.axis_index("s") * 2 + lax.axis_index("c")
        base = wid * rows_per_w
        pltpu.sync_copy(p_hbm.at[pl.ds(base, rows_per_w)], pbuf)

        def chunk_body(c, _):
            row0 = base + c * _CHR
            pltpu.sync_copy(x_hbm.at[pl.ds(row0, _CHR), :], xbuf)

            def row_body(r, _):
                splat = plsc.load_gather(
                    pbuf, [jnp.full((16,), c * _CHR + r, jnp.int32)]
                )

                def vec_body(j, _):
                    sl = pl.ds(j * 16, 16)
                    xbuf[r, sl] = xbuf[r, sl] + splat
                    return 0

                return lax.fori_loop(0, B // 16, vec_body, 0)

            lax.fori_loop(0, _CHR, row_body, 0)
            pltpu.sync_copy(xbuf, o_hbm.at[pl.ds(row0, _CHR), :])
            return 0

        lax.fori_loop(0, n_chunks, chunk_body, 0)

    out2 = sc_add(x2, pos_flat)
    return jnp.transpose(out2.reshape(L, D, B), (2, 0, 1))
